# Initial kernel scaffold; baseline (speedup 1.0000x reference)
import jax
import jax.numpy as jnp
from jax.experimental import pallas as pl


def kernel(x, monomer_labels_i):
    raise NotImplementedError("write your pallas kernel here")


# speedup vs baseline: 2.5676x; 2.5676x over previous
import functools

import jax
import jax.numpy as jnp
from jax import lax
from jax.experimental import pallas as pl
from jax.experimental.pallas import tpu as pltpu
from jax.experimental.pallas import tpu_sc as plsc

N = 320000
D = 128
NUM_SEG = 10000
R = 128
NCHUNKS = N // R
NTILES = 16
SEG_SLICE = 624


def _body(x_hbm, lab_hbm, zero_hbm, out_hbm, lab_v, rows_v, acc_sh):
    t = lax.axis_index("s")

    pltpu.sync_copy(
        zero_hbm.at[pl.ds(t * SEG_SLICE, SEG_SLICE), :],
        acc_sh.at[pl.ds(t * SEG_SLICE, SEG_SLICE), :],
    )

    @pl.when(t == 0)
    def _():
        pltpu.sync_copy(
            zero_hbm.at[pl.ds(NTILES * SEG_SLICE, NUM_SEG - NTILES * SEG_SLICE), :],
            acc_sh.at[pl.ds(NTILES * SEG_SLICE, NUM_SEG - NTILES * SEG_SLICE), :],
        )

    plsc.subcore_barrier()

    n = 156 + jnp.where(t < NCHUNKS - 156 * NTILES, 1, 0)

    def chunk_body(k, carry):
        c = t + NTILES * k
        pltpu.sync_copy(lab_hbm.at[pl.ds(c * R, R)], lab_v)
        pltpu.sync_copy(x_hbm.at[pl.ds(c * R, R), :], rows_v)
        pltpu.sync_copy(rows_v, acc_sh.at[lab_v], add=True)
        return carry

    lax.fori_loop(0, n, chunk_body, 0)
    plsc.subcore_barrier()

    pltpu.sync_copy(
        acc_sh.at[pl.ds(t * SEG_SLICE, SEG_SLICE), :],
        out_hbm.at[pl.ds(t * SEG_SLICE, SEG_SLICE), :],
    )

    @pl.when(t == 0)
    def _():
        pltpu.sync_copy(
            acc_sh.at[pl.ds(NTILES * SEG_SLICE, NUM_SEG - NTILES * SEG_SLICE), :],
            out_hbm.at[pl.ds(NTILES * SEG_SLICE, NUM_SEG - NTILES * SEG_SLICE), :],
        )


@jax.jit
def kernel(x, monomer_labels_i):
    zeros = jnp.zeros((NUM_SEG, D), jnp.float32)
    mesh = plsc.VectorSubcoreMesh(
        core_axis_name="c", subcore_axis_name="s", num_cores=1
    )
    f = pl.kernel(
        _body,
        out_type=jax.ShapeDtypeStruct((NUM_SEG, D), jnp.float32),
        mesh=mesh,
        scratch_types=[
            pltpu.VMEM((R,), jnp.int32),
            pltpu.VMEM((R, D), jnp.float32),
            pltpu.VMEM_SHARED((NUM_SEG, D), jnp.float32),
        ],
    )
    return f(x, monomer_labels_i, zeros)


# speedup vs baseline: 4.7966x; 1.8681x over previous
import functools

import jax
import jax.numpy as jnp
from jax import lax
from jax.experimental import pallas as pl
from jax.experimental.pallas import tpu as pltpu
from jax.experimental.pallas import tpu_sc as plsc

N = 320000
D = 128
NUM_SEG = 10000
R = 128
NCHUNKS = N // R
NTILES = 16
PER_TILE = NCHUNKS // NTILES
PAIRS = PER_TILE // 2
SEG_SLICE = 624


def _body(x_hbm, lab_hbm, zero_hbm, out_hbm,
          lab0, lab1, rows0, rows1, acc_sh,
          sem_r0, sem_r1, sem_l0, sem_l1):
    t = lax.axis_index("s")
    labs = (lab0, lab1)
    rows = (rows0, rows1)
    sem_r = (sem_r0, sem_r1)
    sem_l = (sem_l0, sem_l1)

    pltpu.sync_copy(
        zero_hbm.at[pl.ds(t * SEG_SLICE, SEG_SLICE), :],
        acc_sh.at[pl.ds(t * SEG_SLICE, SEG_SLICE), :],
    )

    @pl.when(t == 0)
    def _():
        pltpu.sync_copy(
            zero_hbm.at[pl.ds(NTILES * SEG_SLICE, NUM_SEG - NTILES * SEG_SLICE), :],
            acc_sh.at[pl.ds(NTILES * SEG_SLICE, NUM_SEG - NTILES * SEG_SLICE), :],
        )

    plsc.subcore_barrier()

    def chunk_of(k):
        return t + NTILES * k

    def issue(b, k):
        c = chunk_of(k)
        pltpu.async_copy(lab_hbm.at[pl.ds(c * R, R)], labs[b], sem_l[b])
        pltpu.async_copy(x_hbm.at[pl.ds(c * R, R), :], rows[b], sem_r[b])

    def wait(b, k):
        c = chunk_of(k)
        pltpu.make_async_copy(lab_hbm.at[pl.ds(c * R, R)], labs[b], sem_l[b]).wait()
        pltpu.make_async_copy(x_hbm.at[pl.ds(c * R, R), :], rows[b], sem_r[b]).wait()

    issue(0, 0)
    issue(1, 1)

    def pair_body(g, carry):
        for b in (0, 1):
            k = 2 * g + b
            wait(b, k)
            pltpu.sync_copy(rows[b], acc_sh.at[labs[b]], add=True)

            @pl.when(k + 2 < PER_TILE)
            def _():
                issue(b, k + 2)
        return carry

    lax.fori_loop(0, PAIRS, pair_body, 0)

    @pl.when(t < NCHUNKS - NTILES * PER_TILE)
    def _():
        c = NTILES * PER_TILE + t
        pltpu.sync_copy(lab_hbm.at[pl.ds(c * R, R)], lab0)
        pltpu.sync_copy(x_hbm.at[pl.ds(c * R, R), :], rows0)
        pltpu.sync_copy(rows0, acc_sh.at[lab0], add=True)

    plsc.subcore_barrier()

    pltpu.sync_copy(
        acc_sh.at[pl.ds(t * SEG_SLICE, SEG_SLICE), :],
        out_hbm.at[pl.ds(t * SEG_SLICE, SEG_SLICE), :],
    )

    @pl.when(t == 0)
    def _():
        pltpu.sync_copy(
            acc_sh.at[pl.ds(NTILES * SEG_SLICE, NUM_SEG - NTILES * SEG_SLICE), :],
            out_hbm.at[pl.ds(NTILES * SEG_SLICE, NUM_SEG - NTILES * SEG_SLICE), :],
        )


@jax.jit
def kernel(x, monomer_labels_i):
    zeros = jnp.zeros((NUM_SEG, D), jnp.float32)
    mesh = plsc.VectorSubcoreMesh(
        core_axis_name="c", subcore_axis_name="s", num_cores=1
    )
    f = pl.kernel(
        _body,
        out_type=jax.ShapeDtypeStruct((NUM_SEG, D), jnp.float32),
        mesh=mesh,
        scratch_types=[
            pltpu.VMEM((R,), jnp.int32),
            pltpu.VMEM((R,), jnp.int32),
            pltpu.VMEM((R, D), jnp.float32),
            pltpu.VMEM((R, D), jnp.float32),
            pltpu.VMEM_SHARED((NUM_SEG, D), jnp.float32),
            pltpu.SemaphoreType.DMA,
            pltpu.SemaphoreType.DMA,
            pltpu.SemaphoreType.DMA,
            pltpu.SemaphoreType.DMA,
        ],
    )
    return f(x, monomer_labels_i, zeros)


# speedup vs baseline: 8.0664x; 1.6817x over previous
import functools

import jax
import jax.numpy as jnp
from jax import lax
from jax.experimental import pallas as pl
from jax.experimental.pallas import tpu as pltpu
from jax.experimental.pallas import tpu_sc as plsc

N = 320000
D = 128
NUM_SEG = 10000
R = 128
NCHUNKS = N // R
NTILES = 16
SEG_SLICE = 624


def _lane(ref, lane):
    return ref[...][lane]


def _body(x_hbm, lab_hbm, zero_hbm, out_hbm,
          lab0, lab1, rows0, rows1, probe_v, acc_sh,
          sem_r0, sem_r1, sem_l0, sem_l1):
    cid = lax.axis_index("c")
    s = lax.axis_index("s")
    labs = (lab0, lab1)
    rows = (rows0, rows1)
    sem_r = (sem_r0, sem_r1)
    sem_l = (sem_l0, sem_l1)

    pltpu.sync_copy(
        zero_hbm.at[pl.ds(s * SEG_SLICE, SEG_SLICE), :],
        acc_sh.at[pl.ds(s * SEG_SLICE, SEG_SLICE), :],
    )

    @pl.when(s == 0)
    def _():
        pltpu.sync_copy(
            zero_hbm.at[pl.ds(NTILES * SEG_SLICE, NUM_SEG - NTILES * SEG_SLICE), :],
            acc_sh.at[pl.ds(NTILES * SEG_SLICE, NUM_SEG - NTILES * SEG_SLICE), :],
        )

    pltpu.sync_copy(lab_hbm.at[pl.ds(N // 2, 16)], probe_v)
    m = _lane(probe_v, 0)
    M = (m // 8 + 1) * 8

    def bs_body(i, lohi):
        lo, hi = lohi
        mid = (lo + hi) // 2
        pltpu.sync_copy(lab_hbm.at[pl.ds((mid + 1) * R - 16, 16)], probe_v)
        last = _lane(probe_v, 15)
        ge = last >= M
        return (jnp.where(ge, lo, mid + 1), jnp.where(ge, mid, hi))

    lo, _ = lax.fori_loop(0, 12, bs_body, (jnp.int32(0), jnp.int32(NCHUNKS)))
    cstar = lo

    start = jnp.where(cid == 0, s, cstar + s)
    limit = jnp.where(cid == 0, jnp.minimum(cstar, NCHUNKS - 1) + 1, NCHUNKS)
    n = jnp.maximum(0, (limit - start + NTILES - 1) // NTILES)

    def chunk_of(k):
        return start + NTILES * k

    def issue(b, k):
        c = chunk_of(k)
        pltpu.async_copy(lab_hbm.at[pl.ds(c * R, R)], labs[b], sem_l[b])
        pltpu.async_copy(x_hbm.at[pl.ds(c * R, R), :], rows[b], sem_r[b])

    def wait(b, k):
        c = chunk_of(k)
        pltpu.make_async_copy(lab_hbm.at[pl.ds(c * R, R)], labs[b], sem_l[b]).wait()
        pltpu.make_async_copy(x_hbm.at[pl.ds(c * R, R), :], rows[b], sem_r[b]).wait()

    @pl.when(n > 0)
    def _():
        issue(0, 0)

    @pl.when(n > 1)
    def _():
        issue(1, 1)

    def pair_body(g, carry):
        for b in (0, 1):
            k = 2 * g + b

            @pl.when(k < n)
            def _():
                wait(b, k)
                pltpu.sync_copy(rows[b], acc_sh.at[labs[b]], add=True)

                @pl.when(k + 2 < n)
                def _():
                    issue(b, k + 2)
        return carry

    lax.fori_loop(0, (n + 1) // 2, pair_body, 0)
    plsc.subcore_barrier()

    base = jnp.where(cid == 0, 0, M)
    count = jnp.where(cid == 0, M, NUM_SEG - M)
    n64 = count // 64
    nblk = jnp.maximum(0, (n64 - s + NTILES - 1) // NTILES)

    def wr_body(j, carry):
        off = base + 64 * (s + NTILES * j)
        pltpu.sync_copy(acc_sh.at[pl.ds(off, 64), :], out_hbm.at[pl.ds(off, 64), :])
        return carry

    lax.fori_loop(0, nblk, wr_body, 0)

    @pl.when(s == 0)
    def _():
        rem8 = (count - 64 * n64) // 8

        def rem_body(r, carry):
            off = base + 64 * n64 + 8 * r
            pltpu.sync_copy(acc_sh.at[pl.ds(off, 8), :], out_hbm.at[pl.ds(off, 8), :])
            return carry

        lax.fori_loop(0, rem8, rem_body, 0)


@jax.jit
def kernel(x, monomer_labels_i):
    zeros = jnp.zeros((NUM_SEG, D), jnp.float32)
    mesh = plsc.VectorSubcoreMesh(core_axis_name="c", subcore_axis_name="s")
    f = pl.kernel(
        _body,
        out_type=jax.ShapeDtypeStruct((NUM_SEG, D), jnp.float32),
        mesh=mesh,
        scratch_types=[
            pltpu.VMEM((R,), jnp.int32),
            pltpu.VMEM((R,), jnp.int32),
            pltpu.VMEM((R, D), jnp.float32),
            pltpu.VMEM((R, D), jnp.float32),
            pltpu.VMEM((16,), jnp.int32),
            pltpu.VMEM_SHARED((NUM_SEG, D), jnp.float32),
            pltpu.SemaphoreType.DMA,
            pltpu.SemaphoreType.DMA,
            pltpu.SemaphoreType.DMA,
            pltpu.SemaphoreType.DMA,
        ],
    )
    return f(x, monomer_labels_i, zeros)
